# Initial kernel scaffold; baseline (speedup 1.0000x reference)
import jax
import jax.numpy as jnp
from jax.experimental import pallas as pl


def kernel(x, table):
    raise NotImplementedError("write your pallas kernel here")


# speedup vs baseline: 1.5600x; 1.5600x over previous
import functools

import jax
import jax.numpy as jnp
from jax import lax
from jax.experimental import pallas as pl
from jax.experimental.pallas import tpu as pltpu
from jax.experimental.pallas import tpu_sc as plsc

_EMBED = 32
_B = 16384 * 26
_NC = 2
_NS = 16
_NW = _NC * _NS
_BPW = _B // _NW
_CHUNK = 1664
_NCHUNK = _BPW // _CHUNK

_mesh = plsc.VectorSubcoreMesh(core_axis_name="c", subcore_axis_name="s")


@functools.partial(
    pl.kernel,
    mesh=_mesh,
    out_type=jax.ShapeDtypeStruct((_B, _EMBED), jnp.float32),
    scratch_types=[
        pltpu.VMEM((_CHUNK,), jnp.int32),
        pltpu.VMEM((_CHUNK, _EMBED), jnp.float32),
        pltpu.SemaphoreType.DMA,
    ],
    compiler_params=pltpu.CompilerParams(use_tc_tiling_on_sc=False),
)
def _gather(idx_hbm, table_hbm, out_hbm, idx_v, rows_v, sem):
    wid = lax.axis_index("s") * _NC + lax.axis_index("c")
    base = wid * _BPW
    for i in range(_NCHUNK):
        off = base + i * _CHUNK
        pltpu.sync_copy(idx_hbm.at[pl.ds(off, _CHUNK)], idx_v)
        pltpu.async_copy(table_hbm.at[idx_v], rows_v, sem).wait()
        pltpu.sync_copy(rows_v, out_hbm.at[pl.ds(off, _CHUNK)])


def kernel(x, table):
    idx = x.reshape(-1).astype(jnp.int32)
    out = _gather(idx, table)
    return out.reshape(x.shape + (_EMBED,))


# speedup vs baseline: 1.5774x; 1.0112x over previous
import functools

import jax
import jax.numpy as jnp
from jax import lax
from jax.experimental import pallas as pl
from jax.experimental.pallas import tpu as pltpu
from jax.experimental.pallas import tpu_sc as plsc

_EMBED = 32
_B = 16384 * 26
_NC = 2
_NS = 16
_NW = _NC * _NS
_BPW = _B // _NW
_CHUNK = 1664
_NCHUNK = _BPW // _CHUNK

_mesh = plsc.VectorSubcoreMesh(core_axis_name="c", subcore_axis_name="s")


@functools.partial(
    pl.kernel,
    mesh=_mesh,
    out_type=jax.ShapeDtypeStruct((_B, _EMBED), jnp.float32),
    scratch_types=[
        pltpu.VMEM((_NCHUNK, _CHUNK), jnp.int32),
        pltpu.VMEM((2, _CHUNK, _EMBED), jnp.float32),
        pltpu.SemaphoreType.DMA,
        pltpu.SemaphoreType.DMA,
    ],
    compiler_params=pltpu.CompilerParams(use_tc_tiling_on_sc=False),
)
def _gather(idx_hbm, table_hbm, out_hbm, idx_v, rows_v, sem0, sem1):
    wid = lax.axis_index("s") * _NC + lax.axis_index("c")
    base = wid * _BPW
    sems = (sem0, sem1)
    pltpu.sync_copy(idx_hbm.at[pl.ds(wid * _NCHUNK, _NCHUNK)], idx_v)
    pltpu.async_copy(table_hbm.at[idx_v.at[0]], rows_v.at[0], sems[0])
    for i in range(_NCHUNK):
        if i + 1 < _NCHUNK:
            pltpu.async_copy(
                table_hbm.at[idx_v.at[i + 1]], rows_v.at[(i + 1) % 2],
                sems[(i + 1) % 2])
        pltpu.make_async_copy(
            table_hbm.at[idx_v.at[i]], rows_v.at[i % 2], sems[i % 2]).wait()
        pltpu.sync_copy(rows_v.at[i % 2],
                        out_hbm.at[pl.ds(base + i * _CHUNK, _CHUNK)])


def kernel(x, table):
    idx = x.reshape(_B // _CHUNK, _CHUNK).astype(jnp.int32)
    out = _gather(idx, table)
    return out.reshape(x.shape + (_EMBED,))
